# baseline (device time: 151755 ns/iter reference)
import jax
import jax.numpy as jnp
from jax import lax
from jax.experimental import pallas as pl
from jax.experimental.pallas import tpu as pltpu

N_DEV = 32
LOG2 = 5
B, SQ, SKV, HQ_LOC, DH = 2, 256, 256, 4, 64
D_MODEL = 512
HD = HQ_LOC * DH
ROWS = B * SQ
CHUNK = ROWS // N_DEV

RS_HALF = [ROWS >> (k + 1) for k in range(LOG2)]
RS_OFF = [0]
for _h in RS_HALF[:-1]:
    RS_OFF.append(RS_OFF[-1] + _h)
RS_BUF_ROWS = sum(RS_HALF)


N_KV_DMA = 4


def kernel(x, Wq, K_ext, V_ext, Wo):
    x2 = x.reshape(ROWS, D_MODEL)
    K2 = K_ext.reshape(B * SKV, 128 * DH)
    V2 = V_ext.reshape(B * SKV, 128 * DH)

    def body(x_ref, wq_ref, k_hbm, v_hbm, wo_ref, out_ref,
             kbuf, vbuf, ctx_ref, acc_ref, rs_buf,
             copy_sems, rs_send, rs_recv, ag_send, ag_recv):
        my = lax.axis_index("i")

        col = pl.multiple_of(my * HD, HD)
        rows_per = (B * SKV) // N_KV_DMA
        copies = []
        for src, dst, base in ((k_hbm, kbuf, 0), (v_hbm, vbuf, N_KV_DMA)):
            for c in range(N_KV_DMA):
                copies.append(pltpu.make_async_copy(
                    src.at[pl.ds(c * rows_per, rows_per), pl.ds(col, HD)],
                    dst.at[pl.ds(c * rows_per, rows_per), :],
                    copy_sems.at[base + c]))
        for cp in copies:
            cp.start()

        q = jnp.dot(x_ref[:, :], wq_ref[:, :], preferred_element_type=jnp.float32)
        for cp in copies:
            cp.wait()

        for b in range(B):
            for h in range(HQ_LOC):
                qbh = q[b * SQ:(b + 1) * SQ, h * DH:(h + 1) * DH]
                kbh = kbuf[b * SKV:(b + 1) * SKV, h * DH:(h + 1) * DH]
                s = lax.dot_general(
                    qbh, kbh, (((1,), (1,)), ((), ())),
                    preferred_element_type=jnp.float32,
                ) * 0.125
                rb = lax.broadcasted_iota(jnp.int32, (SQ, SKV), 0) // 64
                cb = lax.broadcasted_iota(jnp.int32, (SQ, SKV), 1) // 64
                s = jnp.where(cb <= rb, s, -1e9)
                m = jnp.max(s, axis=1, keepdims=True)
                e = jnp.exp(s - m)
                w = e / jnp.sum(e, axis=1, keepdims=True)
                ctx_ref[b * SQ:(b + 1) * SQ, h * DH:(h + 1) * DH] = jnp.dot(
                    w, vbuf[b * SKV:(b + 1) * SKV, h * DH:(h + 1) * DH],
                    preferred_element_type=jnp.float32)
        acc_ref[:, :] = jnp.dot(ctx_ref[:, :], wo_ref[:, :],
                                preferred_element_type=jnp.float32)

        bar = pltpu.get_barrier_semaphore()
        for k in range(LOG2):
            pl.semaphore_signal(bar, inc=1, device_id=(my ^ (1 << k),),
                                device_id_type=pl.DeviceIdType.MESH)
        pl.semaphore_wait(bar, LOG2)

        lo = my * 0
        for k in range(LOG2):
            half = RS_HALF[k]
            bit = (my >> k) & 1
            keep_lo = pl.multiple_of(lo + bit * half, CHUNK)
            send_lo = pl.multiple_of(lo + (1 - bit) * half, CHUNK)
            rdma = pltpu.make_async_remote_copy(
                src_ref=acc_ref.at[pl.ds(send_lo, half), :],
                dst_ref=rs_buf.at[pl.ds(RS_OFF[k], half), :],
                send_sem=rs_send.at[k],
                recv_sem=rs_recv.at[k],
                device_id=(my ^ (1 << k),),
                device_id_type=pl.DeviceIdType.MESH,
            )
            rdma.start()
            rdma.wait()
            acc_ref[pl.ds(keep_lo, half), :] = (
                acc_ref[pl.ds(keep_lo, half), :]
                + rs_buf[pl.ds(RS_OFF[k], half), :]
            )
            lo = keep_lo
        out_ref[pl.ds(lo, CHUNK), :] = acc_ref[pl.ds(lo, CHUNK), :]

        for idx, j in enumerate(range(LOG2 - 1, -1, -1)):
            size = CHUNK << (LOG2 - 1 - j)
            glo = pl.multiple_of(lo & ~(size - 1), size)
            rdma = pltpu.make_async_remote_copy(
                src_ref=out_ref.at[pl.ds(glo, size), :],
                dst_ref=out_ref.at[pl.ds(glo, size), :],
                send_sem=ag_send.at[idx],
                recv_sem=ag_recv.at[idx],
                device_id=(my ^ (1 << j),),
                device_id_type=pl.DeviceIdType.MESH,
            )
            rdma.start()
            rdma.wait()

    out = pl.pallas_call(
        body,
        out_shape=jax.ShapeDtypeStruct((ROWS, D_MODEL), jnp.float32),
        in_specs=[
            pl.BlockSpec(memory_space=pltpu.VMEM),
            pl.BlockSpec(memory_space=pltpu.VMEM),
            pl.BlockSpec(memory_space=pltpu.HBM),
            pl.BlockSpec(memory_space=pltpu.HBM),
            pl.BlockSpec(memory_space=pltpu.VMEM),
        ],
        out_specs=pl.BlockSpec(memory_space=pltpu.VMEM),
        scratch_shapes=[
            pltpu.VMEM((B * SKV, HD), jnp.float32),
            pltpu.VMEM((B * SKV, HD), jnp.float32),
            pltpu.VMEM((ROWS, HD), jnp.float32),
            pltpu.VMEM((ROWS, D_MODEL), jnp.float32),
            pltpu.VMEM((RS_BUF_ROWS, D_MODEL), jnp.float32),
            pltpu.SemaphoreType.DMA((2 * N_KV_DMA,)),
            pltpu.SemaphoreType.DMA((LOG2,)),
            pltpu.SemaphoreType.DMA((LOG2,)),
            pltpu.SemaphoreType.DMA((LOG2,)),
            pltpu.SemaphoreType.DMA((LOG2,)),
        ],
        compiler_params=pltpu.CompilerParams(collective_id=0),
    )(x2, Wq, K2, V2, Wo)
    return out.reshape(B, SQ, D_MODEL)


# device time: 75885 ns/iter; 1.9998x vs baseline; 1.9998x over previous
import jax
import jax.numpy as jnp
from jax import lax
from jax.experimental import pallas as pl
from jax.experimental.pallas import tpu as pltpu

N_DEV = 32
LOG2 = 5
B, SQ, SKV, HQ_LOC, DH = 2, 256, 256, 4, 64
D_MODEL = 512
HD = HQ_LOC * DH
ROWS = B * SQ
CHUNK = ROWS // N_DEV

RS_HALF = [ROWS >> (k + 1) for k in range(LOG2)]
RS_OFF = [0]
for _h in RS_HALF[:-1]:
    RS_OFF.append(RS_OFF[-1] + _h)
RS_BUF_ROWS = sum(RS_HALF)


def kernel(x, Wq, K_ext, V_ext, Wo):
    i = lax.axis_index("i")
    K_t = jnp.transpose(
        lax.dynamic_slice(K_ext, (0, 0, i * HQ_LOC, 0), (B, SKV, HQ_LOC, DH)),
        (0, 2, 3, 1)).astype(jnp.bfloat16).reshape(B, HD, SKV)
    V_t = jnp.transpose(
        lax.dynamic_slice(V_ext, (0, 0, i * HQ_LOC, 0), (B, SKV, HQ_LOC, DH)),
        (0, 2, 3, 1)).astype(jnp.bfloat16).reshape(B, HD, SKV)

    def body(x_ref, wq_ref, kbuf, vbuf, wo_ref, out_ref,
             ctx_ref, acc_ref, rs_buf, gbuf,
             rs_send, rs_recv, ag_send, ag_recv):
        my = lax.axis_index("i")

        bar = pltpu.get_barrier_semaphore()
        for k in range(LOG2):
            pl.semaphore_signal(bar, inc=1, device_id=(my ^ (1 << k),),
                                device_id_type=pl.DeviceIdType.MESH)

        for b in range(B):
            qb = jnp.dot(x_ref[b], wq_ref[:, :],
                         preferred_element_type=jnp.float32)
            for h in range(HQ_LOC):
                qbh = qb[:, h * DH:(h + 1) * DH].astype(jnp.bfloat16)
                kbh_t = kbuf[b, h * DH:(h + 1) * DH, :]
                s = lax.dot_general(
                    qbh, kbh_t, (((1,), (0,)), ((), ())),
                    preferred_element_type=jnp.float32,
                ) * 0.125
                rb = lax.broadcasted_iota(jnp.int32, (SQ, SKV), 0) // 64
                cb = lax.broadcasted_iota(jnp.int32, (SQ, SKV), 1) // 64
                s = jnp.where(cb <= rb, s, -1e9)
                e = jnp.exp(s)
                w = (e / jnp.sum(e, axis=1, keepdims=True)).astype(jnp.bfloat16)
                ctx_ref[b * SQ:(b + 1) * SQ, h * DH:(h + 1) * DH] = (
                    lax.dot_general(
                        w, vbuf[b, h * DH:(h + 1) * DH, :],
                        (((1,), (1,)), ((), ())),
                        preferred_element_type=jnp.float32))

        bit0 = my & 1
        half0 = RS_HALF[0]
        keep_lo = pl.multiple_of(bit0 * half0, half0)
        send_lo = pl.multiple_of((1 - bit0) * half0, half0)
        acc_ref[pl.ds(send_lo, half0), :] = jnp.dot(
            ctx_ref[pl.ds(send_lo, half0), :], wo_ref[:, :],
            preferred_element_type=jnp.float32)
        pl.semaphore_wait(bar, LOG2)
        rdma0 = pltpu.make_async_remote_copy(
            src_ref=acc_ref.at[pl.ds(send_lo, half0), :],
            dst_ref=rs_buf.at[pl.ds(RS_OFF[0], half0), :],
            send_sem=rs_send.at[0],
            recv_sem=rs_recv.at[0],
            device_id=(my ^ 1,),
            device_id_type=pl.DeviceIdType.MESH,
        )
        rdma0.start()
        acc_ref[pl.ds(keep_lo, half0), :] = jnp.dot(
            ctx_ref[pl.ds(keep_lo, half0), :], wo_ref[:, :],
            preferred_element_type=jnp.float32)

        rdma0.wait_recv()
        acc_ref[pl.ds(keep_lo, half0), :] = (
            acc_ref[pl.ds(keep_lo, half0), :]
            + rs_buf[pl.ds(RS_OFF[0], half0), :]
        )
        lo = keep_lo
        send_waits = [rdma0]
        for k in range(1, LOG2):
            half = RS_HALF[k]
            bit = (my >> k) & 1
            keep_lo = pl.multiple_of(lo + bit * half, CHUNK)
            send_lo = pl.multiple_of(lo + (1 - bit) * half, CHUNK)
            rdma = pltpu.make_async_remote_copy(
                src_ref=acc_ref.at[pl.ds(send_lo, half), :],
                dst_ref=rs_buf.at[pl.ds(RS_OFF[k], half), :],
                send_sem=rs_send.at[k],
                recv_sem=rs_recv.at[k],
                device_id=(my ^ (1 << k),),
                device_id_type=pl.DeviceIdType.MESH,
            )
            rdma.start()
            rdma.wait_recv()
            acc_ref[pl.ds(keep_lo, half), :] = (
                acc_ref[pl.ds(keep_lo, half), :]
                + rs_buf[pl.ds(RS_OFF[k], half), :]
            )
            lo = keep_lo
            send_waits.append(rdma)
        gbuf[pl.ds(lo, CHUNK), :] = acc_ref[pl.ds(lo, CHUNK), :]

        for idx, j in enumerate(range(LOG2 - 1, -1, -1)):
            size = CHUNK << (LOG2 - 1 - j)
            glo = pl.multiple_of(lo & ~(size - 1), size)
            rdma = pltpu.make_async_remote_copy(
                src_ref=gbuf.at[pl.ds(glo, size), :],
                dst_ref=gbuf.at[pl.ds(glo, size), :],
                send_sem=ag_send.at[idx],
                recv_sem=ag_recv.at[idx],
                device_id=(my ^ (1 << j),),
                device_id_type=pl.DeviceIdType.MESH,
            )
            rdma.start()
            rdma.wait_recv()
            send_waits.append(rdma)

        for b in range(B):
            out_ref[b] = gbuf[b * SQ:(b + 1) * SQ, :]
        for rd in send_waits:
            rd.wait_send()

    out = pl.pallas_call(
        body,
        out_shape=jax.ShapeDtypeStruct((B, SQ, D_MODEL), jnp.float32),
        in_specs=[pl.BlockSpec(memory_space=pltpu.VMEM)] * 5,
        out_specs=pl.BlockSpec(memory_space=pltpu.VMEM),
        scratch_shapes=[
            pltpu.VMEM((ROWS, HD), jnp.float32),
            pltpu.VMEM((ROWS, D_MODEL), jnp.float32),
            pltpu.VMEM((RS_BUF_ROWS, D_MODEL), jnp.float32),
            pltpu.VMEM((ROWS, D_MODEL), jnp.float32),
            pltpu.SemaphoreType.DMA((LOG2,)),
            pltpu.SemaphoreType.DMA((LOG2,)),
            pltpu.SemaphoreType.DMA((LOG2,)),
            pltpu.SemaphoreType.DMA((LOG2,)),
        ],
        compiler_params=pltpu.CompilerParams(collective_id=0),
    )(x, Wq, K_t, V_t, Wo)
    return out


# device time: 64523 ns/iter; 2.3520x vs baseline; 1.1761x over previous
import jax
import jax.numpy as jnp
from jax import lax
from jax.experimental import pallas as pl
from jax.experimental.pallas import tpu as pltpu

N_DEV = 32
LOG2 = 5
B, SQ, SKV, HQ_LOC, DH = 2, 256, 256, 4, 64
D_MODEL = 512
HD = HQ_LOC * DH
ROWS = B * SQ
CHUNK = ROWS // N_DEV

RS_HALF = [ROWS >> (k + 1) for k in range(LOG2)]
RS_OFF = [0]
for _h in RS_HALF[:-1]:
    RS_OFF.append(RS_OFF[-1] + _h)
RS_BUF_ROWS = sum(RS_HALF)


def kernel(x, Wq, K_ext, V_ext, Wo):
    i = lax.axis_index("i")
    K_t = jnp.transpose(
        lax.dynamic_slice(K_ext, (0, 0, i * HQ_LOC, 0), (B, SKV, HQ_LOC, DH)),
        (0, 2, 3, 1)).astype(jnp.bfloat16).reshape(B, HD, SKV)
    V_t = jnp.transpose(
        lax.dynamic_slice(V_ext, (0, 0, i * HQ_LOC, 0), (B, SKV, HQ_LOC, DH)),
        (0, 2, 3, 1)).astype(jnp.bfloat16).reshape(B, HD, SKV)

    def body(x_ref, wq_ref, kbuf, vbuf, wo_ref, out_ref,
             ctx_ref, acc_ref, rs_buf, gbuf,
             rs_send, rs_recv, ag_send, ag_recv):
        my = lax.axis_index("i")

        bar = pltpu.get_barrier_semaphore()
        for k in range(LOG2):
            pl.semaphore_signal(bar, inc=1, device_id=(my ^ (1 << k),),
                                device_id_type=pl.DeviceIdType.MESH)

        wq_bf = wq_ref[:, :].astype(jnp.bfloat16)
        wo_bf = wo_ref[:, :].astype(jnp.bfloat16)
        for b in range(B):
            qb = jnp.dot(x_ref[b].astype(jnp.bfloat16), wq_bf,
                         preferred_element_type=jnp.float32)
            for h in range(HQ_LOC):
                qbh = qb[:, h * DH:(h + 1) * DH].astype(jnp.bfloat16)
                kbh_t = kbuf[b, h * DH:(h + 1) * DH, :]
                s = lax.dot_general(
                    qbh, kbh_t, (((1,), (0,)), ((), ())),
                    preferred_element_type=jnp.float32,
                ) * 0.125
                rb = lax.broadcasted_iota(jnp.int32, (SQ, SKV), 0) // 64
                cb = lax.broadcasted_iota(jnp.int32, (SQ, SKV), 1) // 64
                s = jnp.where(cb <= rb, s, -1e9)
                e = jnp.exp(s)
                w = (e / jnp.sum(e, axis=1, keepdims=True)).astype(jnp.bfloat16)
                ctx_ref[b * SQ:(b + 1) * SQ, h * DH:(h + 1) * DH] = (
                    lax.dot_general(
                        w, vbuf[b, h * DH:(h + 1) * DH, :],
                        (((1,), (1,)), ((), ())),
                        preferred_element_type=jnp.float32)
                    .astype(jnp.bfloat16))

        bit0 = my & 1
        half0 = RS_HALF[0]
        keep_lo = pl.multiple_of(bit0 * half0, half0)
        send_lo = pl.multiple_of((1 - bit0) * half0, half0)
        acc_ref[pl.ds(send_lo, half0), :] = jnp.dot(
            ctx_ref[pl.ds(send_lo, half0), :], wo_bf,
            preferred_element_type=jnp.float32).astype(jnp.bfloat16)
        pl.semaphore_wait(bar, LOG2)
        rdma0 = pltpu.make_async_remote_copy(
            src_ref=acc_ref.at[pl.ds(send_lo, half0), :],
            dst_ref=rs_buf.at[pl.ds(RS_OFF[0], half0), :],
            send_sem=rs_send.at[0],
            recv_sem=rs_recv.at[0],
            device_id=(my ^ 1,),
            device_id_type=pl.DeviceIdType.MESH,
        )
        rdma0.start()
        acc_ref[pl.ds(keep_lo, half0), :] = jnp.dot(
            ctx_ref[pl.ds(keep_lo, half0), :], wo_bf,
            preferred_element_type=jnp.float32).astype(jnp.bfloat16)

        rdma0.wait_recv()
        acc_ref[pl.ds(keep_lo, half0), :] = (
            acc_ref[pl.ds(keep_lo, half0), :]
            + rs_buf[pl.ds(RS_OFF[0], half0), :]
        )
        lo = keep_lo
        send_waits = [rdma0]
        for k in range(1, LOG2):
            half = RS_HALF[k]
            bit = (my >> k) & 1
            keep_lo = pl.multiple_of(lo + bit * half, CHUNK)
            send_lo = pl.multiple_of(lo + (1 - bit) * half, CHUNK)
            rdma = pltpu.make_async_remote_copy(
                src_ref=acc_ref.at[pl.ds(send_lo, half), :],
                dst_ref=rs_buf.at[pl.ds(RS_OFF[k], half), :],
                send_sem=rs_send.at[k],
                recv_sem=rs_recv.at[k],
                device_id=(my ^ (1 << k),),
                device_id_type=pl.DeviceIdType.MESH,
            )
            rdma.start()
            rdma.wait_recv()
            acc_ref[pl.ds(keep_lo, half), :] = (
                acc_ref[pl.ds(keep_lo, half), :]
                + rs_buf[pl.ds(RS_OFF[k], half), :]
            )
            lo = keep_lo
            send_waits.append(rdma)
        gbuf[pl.ds(lo, CHUNK), :] = acc_ref[pl.ds(lo, CHUNK), :]

        for idx, j in enumerate(range(LOG2 - 1, -1, -1)):
            size = CHUNK << (LOG2 - 1 - j)
            glo = pl.multiple_of(lo & ~(size - 1), size)
            rdma = pltpu.make_async_remote_copy(
                src_ref=gbuf.at[pl.ds(glo, size), :],
                dst_ref=gbuf.at[pl.ds(glo, size), :],
                send_sem=ag_send.at[idx],
                recv_sem=ag_recv.at[idx],
                device_id=(my ^ (1 << j),),
                device_id_type=pl.DeviceIdType.MESH,
            )
            rdma.start()
            rdma.wait_recv()
            send_waits.append(rdma)

        for b in range(B):
            out_ref[b] = gbuf[b * SQ:(b + 1) * SQ, :].astype(jnp.float32)
        for rd in send_waits:
            rd.wait_send()

    out = pl.pallas_call(
        body,
        out_shape=jax.ShapeDtypeStruct((B, SQ, D_MODEL), jnp.float32),
        in_specs=[pl.BlockSpec(memory_space=pltpu.VMEM)] * 5,
        out_specs=pl.BlockSpec(memory_space=pltpu.VMEM),
        scratch_shapes=[
            pltpu.VMEM((ROWS, HD), jnp.bfloat16),
            pltpu.VMEM((ROWS, D_MODEL), jnp.bfloat16),
            pltpu.VMEM((RS_BUF_ROWS, D_MODEL), jnp.bfloat16),
            pltpu.VMEM((ROWS, D_MODEL), jnp.bfloat16),
            pltpu.SemaphoreType.DMA((LOG2,)),
            pltpu.SemaphoreType.DMA((LOG2,)),
            pltpu.SemaphoreType.DMA((LOG2,)),
            pltpu.SemaphoreType.DMA((LOG2,)),
        ],
        compiler_params=pltpu.CompilerParams(collective_id=0),
    )(x, Wq, K_t, V_t, Wo)
    return out


# device time: 61805 ns/iter; 2.4554x vs baseline; 1.0440x over previous
import jax
import jax.numpy as jnp
from jax import lax
from jax.experimental import pallas as pl
from jax.experimental.pallas import tpu as pltpu

N_DEV = 32
LOG2 = 5
B, SQ, SKV, HQ_LOC, DH = 2, 256, 256, 4, 64
D_MODEL = 512
HD = HQ_LOC * DH
ROWS = B * SQ
CHUNK = ROWS // N_DEV

RS_HALF = [ROWS >> (k + 1) for k in range(LOG2)]
RS_OFF = [0]
for _h in RS_HALF[:-1]:
    RS_OFF.append(RS_OFF[-1] + _h)
RS_BUF_ROWS = sum(RS_HALF)


def kernel(x, Wq, K_ext, V_ext, Wo):
    i = lax.axis_index("i")
    K_t = jnp.transpose(
        lax.dynamic_slice(K_ext, (0, 0, i * HQ_LOC, 0), (B, SKV, HQ_LOC, DH)),
        (0, 2, 3, 1)).astype(jnp.bfloat16).reshape(B, HD, SKV)
    V_t = jnp.transpose(
        lax.dynamic_slice(V_ext, (0, 0, i * HQ_LOC, 0), (B, SKV, HQ_LOC, DH)),
        (0, 2, 3, 1)).astype(jnp.bfloat16).reshape(B, HD, SKV)

    def body(x_ref, wq_ref, kbuf, vbuf, wo_ref, out_ref,
             ctx_ref, acc_ref, rs_buf, rs8_buf, gbuf,
             rs_send, rs_recv, rs8_send, rs8_recv,
             ag8_send, ag8_recv, ag_send, ag_recv):
        my = lax.axis_index("i")

        peers = [1, 2] + [c << 2 for c in range(1, 8)]
        bar = pltpu.get_barrier_semaphore()
        for d in peers:
            pl.semaphore_signal(bar, inc=1, device_id=(my ^ d,),
                                device_id_type=pl.DeviceIdType.MESH)

        wq_bf = wq_ref[:, :].astype(jnp.bfloat16)
        wo_bf = wo_ref[:, :].astype(jnp.bfloat16)
        for b in range(B):
            qb = jnp.dot(x_ref[b].astype(jnp.bfloat16), wq_bf,
                         preferred_element_type=jnp.float32)
            for h in range(HQ_LOC):
                qbh = qb[:, h * DH:(h + 1) * DH].astype(jnp.bfloat16)
                kbh_t = kbuf[b, h * DH:(h + 1) * DH, :]
                s = lax.dot_general(
                    qbh, kbh_t, (((1,), (0,)), ((), ())),
                    preferred_element_type=jnp.float32,
                ) * 0.125
                rb = lax.broadcasted_iota(jnp.int32, (SQ, SKV), 0) // 64
                cb = lax.broadcasted_iota(jnp.int32, (SQ, SKV), 1) // 64
                s = jnp.where(cb <= rb, s, -1e9)
                e = jnp.exp(s)
                w = (e / jnp.sum(e, axis=1, keepdims=True)).astype(jnp.bfloat16)
                ctx_ref[b * SQ:(b + 1) * SQ, h * DH:(h + 1) * DH] = (
                    lax.dot_general(
                        w, vbuf[b, h * DH:(h + 1) * DH, :],
                        (((1,), (1,)), ((), ())),
                        preferred_element_type=jnp.float32)
                    .astype(jnp.bfloat16))

        bit0 = my & 1
        half0 = RS_HALF[0]
        keep_lo = pl.multiple_of(bit0 * half0, half0)
        send_lo = pl.multiple_of((1 - bit0) * half0, half0)
        acc_ref[pl.ds(send_lo, half0), :] = jnp.dot(
            ctx_ref[pl.ds(send_lo, half0), :], wo_bf,
            preferred_element_type=jnp.float32).astype(jnp.bfloat16)
        pl.semaphore_wait(bar, len(peers))
        rdma0 = pltpu.make_async_remote_copy(
            src_ref=acc_ref.at[pl.ds(send_lo, half0), :],
            dst_ref=rs_buf.at[pl.ds(RS_OFF[0], half0), :],
            send_sem=rs_send.at[0],
            recv_sem=rs_recv.at[0],
            device_id=(my ^ 1,),
            device_id_type=pl.DeviceIdType.MESH,
        )
        rdma0.start()
        acc_ref[pl.ds(keep_lo, half0), :] = jnp.dot(
            ctx_ref[pl.ds(keep_lo, half0), :], wo_bf,
            preferred_element_type=jnp.float32).astype(jnp.bfloat16)

        rdma0.wait_recv()
        acc_ref[pl.ds(keep_lo, half0), :] = (
            acc_ref[pl.ds(keep_lo, half0), :]
            + rs_buf[pl.ds(RS_OFF[0], half0), :]
        )
        lo = keep_lo
        send_waits = [rdma0]
        half = RS_HALF[1]
        bit = (my >> 1) & 1
        keep_lo = pl.multiple_of(lo + bit * half, CHUNK)
        send_lo = pl.multiple_of(lo + (1 - bit) * half, CHUNK)
        rdma = pltpu.make_async_remote_copy(
            src_ref=acc_ref.at[pl.ds(send_lo, half), :],
            dst_ref=rs_buf.at[pl.ds(RS_OFF[1], half), :],
            send_sem=rs_send.at[1],
            recv_sem=rs_recv.at[1],
            device_id=(my ^ 2,),
            device_id_type=pl.DeviceIdType.MESH,
        )
        rdma.start()
        rdma.wait_recv()
        acc_ref[pl.ds(keep_lo, half), :] = (
            acc_ref[pl.ds(keep_lo, half), :]
            + rs_buf[pl.ds(RS_OFF[1], half), :]
        )
        lo2 = keep_lo
        send_waits.append(rdma)

        g_me = (my >> 2) & 7
        flo = pl.multiple_of(lo2 + g_me * CHUNK, CHUNK)
        rs8 = []
        for c in range(1, 8):
            sc = pl.multiple_of(lo2 + ((g_me ^ c) * CHUNK), CHUNK)
            r = pltpu.make_async_remote_copy(
                src_ref=acc_ref.at[pl.ds(sc, CHUNK), :],
                dst_ref=rs8_buf.at[c - 1],
                send_sem=rs8_send.at[c - 1],
                recv_sem=rs8_recv.at[c - 1],
                device_id=(my ^ (c << 2),),
                device_id_type=pl.DeviceIdType.MESH,
            )
            r.start()
            rs8.append(r)
        acc_f = acc_ref[pl.ds(flo, CHUNK), :]
        for c in range(1, 8):
            rs8[c - 1].wait_recv()
            acc_f = acc_f + rs8_buf[c - 1]
        send_waits.extend(rs8)
        gbuf[pl.ds(flo, CHUNK), :] = acc_f

        ag8 = []
        for c in range(1, 8):
            r = pltpu.make_async_remote_copy(
                src_ref=gbuf.at[pl.ds(flo, CHUNK), :],
                dst_ref=gbuf.at[pl.ds(flo, CHUNK), :],
                send_sem=ag8_send.at[c - 1],
                recv_sem=ag8_recv.at[c - 1],
                device_id=(my ^ (c << 2),),
                device_id_type=pl.DeviceIdType.MESH,
            )
            r.start()
            ag8.append(r)
        for r in ag8:
            r.wait_recv()
        send_waits.extend(ag8)

        for idx, j in enumerate((1, 0)):
            size = RS_HALF[j]
            glo = pl.multiple_of(flo & ~(size - 1), size)
            rdma = pltpu.make_async_remote_copy(
                src_ref=gbuf.at[pl.ds(glo, size), :],
                dst_ref=gbuf.at[pl.ds(glo, size), :],
                send_sem=ag_send.at[idx],
                recv_sem=ag_recv.at[idx],
                device_id=(my ^ (1 << j),),
                device_id_type=pl.DeviceIdType.MESH,
            )
            rdma.start()
            rdma.wait_recv()
            send_waits.append(rdma)

        for b in range(B):
            out_ref[b] = gbuf[b * SQ:(b + 1) * SQ, :].astype(jnp.float32)
        for rd in send_waits:
            rd.wait_send()

    out = pl.pallas_call(
        body,
        out_shape=jax.ShapeDtypeStruct((B, SQ, D_MODEL), jnp.float32),
        in_specs=[pl.BlockSpec(memory_space=pltpu.VMEM)] * 5,
        out_specs=pl.BlockSpec(memory_space=pltpu.VMEM),
        scratch_shapes=[
            pltpu.VMEM((ROWS, HD), jnp.bfloat16),
            pltpu.VMEM((ROWS, D_MODEL), jnp.bfloat16),
            pltpu.VMEM((RS_BUF_ROWS, D_MODEL), jnp.bfloat16),
            pltpu.VMEM((7, CHUNK, D_MODEL), jnp.bfloat16),
            pltpu.VMEM((ROWS, D_MODEL), jnp.bfloat16),
            pltpu.SemaphoreType.DMA((2,)),
            pltpu.SemaphoreType.DMA((2,)),
            pltpu.SemaphoreType.DMA((7,)),
            pltpu.SemaphoreType.DMA((7,)),
            pltpu.SemaphoreType.DMA((7,)),
            pltpu.SemaphoreType.DMA((7,)),
            pltpu.SemaphoreType.DMA((2,)),
            pltpu.SemaphoreType.DMA((2,)),
        ],
        compiler_params=pltpu.CompilerParams(collective_id=0),
    )(x, Wq, K_t, V_t, Wo)
    return out


# device time: 49962 ns/iter; 3.0374x vs baseline; 1.2370x over previous
import jax
import jax.numpy as jnp
from jax import lax
from jax.experimental import pallas as pl
from jax.experimental.pallas import tpu as pltpu

N_DEV = 32
LOG2 = 5
B, SQ, SKV, HQ_LOC, DH = 2, 256, 256, 4, 64
D_MODEL = 512
HD = HQ_LOC * DH
ROWS = B * SQ
CHUNK = ROWS // N_DEV

RS_HALF = [ROWS >> (k + 1) for k in range(LOG2)]
RS_OFF = [0]
for _h in RS_HALF[:-1]:
    RS_OFF.append(RS_OFF[-1] + _h)
RS_BUF_ROWS = sum(RS_HALF)


def kernel(x, Wq, K_ext, V_ext, Wo):
    i = lax.axis_index("i")
    K_s = lax.dynamic_slice(
        jnp.transpose(K_ext, (0, 1, 3, 2)), (0, 0, 0, i * HQ_LOC),
        (B, SKV, DH, HQ_LOC)).astype(jnp.bfloat16)
    V_s = lax.dynamic_slice(
        jnp.transpose(V_ext, (0, 1, 3, 2)), (0, 0, 0, i * HQ_LOC),
        (B, SKV, DH, HQ_LOC)).astype(jnp.bfloat16)

    def body(x_ref, wq_ref, kbuf, vbuf, wo_ref, out_ref,
             ctx_ref, acc_ref, rs_buf, rs8_buf, gbuf,
             rs_send, rs_recv, rs8_send, rs8_recv,
             ag8_send, ag8_recv, ag_send, ag_recv):
        my = lax.axis_index("i")

        peers = [1, 2] + [c << 2 for c in range(1, 8)]
        bar = pltpu.get_barrier_semaphore()
        for d in peers:
            pl.semaphore_signal(bar, inc=1, device_id=(my ^ d,),
                                device_id_type=pl.DeviceIdType.MESH)

        kmat = [jnp.transpose(kbuf[b], (0, 2, 1)).reshape(SKV, HD)
                for b in range(B)]
        vmat = [jnp.transpose(vbuf[b], (0, 2, 1)).reshape(SKV, HD)
                for b in range(B)]

        wq_bf = wq_ref[:, :].astype(jnp.bfloat16)
        wo_bf = wo_ref[:, :].astype(jnp.bfloat16)
        for b in range(B):
            qb = jnp.dot(x_ref[b].astype(jnp.bfloat16), wq_bf,
                         preferred_element_type=jnp.float32)
            for h in range(HQ_LOC):
                qbh = qb[:, h * DH:(h + 1) * DH].astype(jnp.bfloat16)
                kbh = kmat[b][:, h * DH:(h + 1) * DH]
                s = lax.dot_general(
                    qbh, kbh, (((1,), (1,)), ((), ())),
                    preferred_element_type=jnp.float32,
                ) * 0.125
                rb = lax.broadcasted_iota(jnp.int32, (SQ, SKV), 0) // 64
                cb = lax.broadcasted_iota(jnp.int32, (SQ, SKV), 1) // 64
                s = jnp.where(cb <= rb, s, -1e9)
                e = jnp.exp(s)
                w = (e / jnp.sum(e, axis=1, keepdims=True)).astype(jnp.bfloat16)
                ctx_ref[b * SQ:(b + 1) * SQ, h * DH:(h + 1) * DH] = (
                    jnp.dot(w, vmat[b][:, h * DH:(h + 1) * DH],
                            preferred_element_type=jnp.float32)
                    .astype(jnp.bfloat16))

        bit0 = my & 1
        half0 = RS_HALF[0]
        keep_lo = pl.multiple_of(bit0 * half0, half0)
        send_lo = pl.multiple_of((1 - bit0) * half0, half0)
        acc_ref[pl.ds(send_lo, half0), :] = jnp.dot(
            ctx_ref[pl.ds(send_lo, half0), :], wo_bf,
            preferred_element_type=jnp.float32).astype(jnp.bfloat16)
        pl.semaphore_wait(bar, len(peers))
        rdma0 = pltpu.make_async_remote_copy(
            src_ref=acc_ref.at[pl.ds(send_lo, half0), :],
            dst_ref=rs_buf.at[pl.ds(RS_OFF[0], half0), :],
            send_sem=rs_send.at[0],
            recv_sem=rs_recv.at[0],
            device_id=(my ^ 1,),
            device_id_type=pl.DeviceIdType.MESH,
        )
        rdma0.start()
        acc_ref[pl.ds(keep_lo, half0), :] = jnp.dot(
            ctx_ref[pl.ds(keep_lo, half0), :], wo_bf,
            preferred_element_type=jnp.float32).astype(jnp.bfloat16)

        rdma0.wait_recv()
        acc_ref[pl.ds(keep_lo, half0), :] = (
            acc_ref[pl.ds(keep_lo, half0), :]
            + rs_buf[pl.ds(RS_OFF[0], half0), :]
        )
        lo = keep_lo
        send_waits = [rdma0]
        half = RS_HALF[1]
        bit = (my >> 1) & 1
        keep_lo = pl.multiple_of(lo + bit * half, CHUNK)
        send_lo = pl.multiple_of(lo + (1 - bit) * half, CHUNK)
        rdma = pltpu.make_async_remote_copy(
            src_ref=acc_ref.at[pl.ds(send_lo, half), :],
            dst_ref=rs_buf.at[pl.ds(RS_OFF[1], half), :],
            send_sem=rs_send.at[1],
            recv_sem=rs_recv.at[1],
            device_id=(my ^ 2,),
            device_id_type=pl.DeviceIdType.MESH,
        )
        rdma.start()
        rdma.wait_recv()
        acc_ref[pl.ds(keep_lo, half), :] = (
            acc_ref[pl.ds(keep_lo, half), :]
            + rs_buf[pl.ds(RS_OFF[1], half), :]
        )
        lo2 = keep_lo
        send_waits.append(rdma)

        g_me = (my >> 2) & 7
        flo = pl.multiple_of(lo2 + g_me * CHUNK, CHUNK)
        rs8 = []
        for c in range(1, 8):
            sc = pl.multiple_of(lo2 + ((g_me ^ c) * CHUNK), CHUNK)
            r = pltpu.make_async_remote_copy(
                src_ref=acc_ref.at[pl.ds(sc, CHUNK), :],
                dst_ref=rs8_buf.at[c - 1],
                send_sem=rs8_send.at[c - 1],
                recv_sem=rs8_recv.at[c - 1],
                device_id=(my ^ (c << 2),),
                device_id_type=pl.DeviceIdType.MESH,
            )
            r.start()
            rs8.append(r)
        acc_f = acc_ref[pl.ds(flo, CHUNK), :]
        for c in range(1, 8):
            rs8[c - 1].wait_recv()
            acc_f = acc_f + rs8_buf[c - 1]
        send_waits.extend(rs8)
        gbuf[pl.ds(flo, CHUNK), :] = acc_f

        ag8 = []
        for c in range(1, 8):
            r = pltpu.make_async_remote_copy(
                src_ref=gbuf.at[pl.ds(flo, CHUNK), :],
                dst_ref=gbuf.at[pl.ds(flo, CHUNK), :],
                send_sem=ag8_send.at[c - 1],
                recv_sem=ag8_recv.at[c - 1],
                device_id=(my ^ (c << 2),),
                device_id_type=pl.DeviceIdType.MESH,
            )
            r.start()
            ag8.append(r)
        for r in ag8:
            r.wait_recv()
        send_waits.extend(ag8)

        for idx, j in enumerate((1, 0)):
            size = RS_HALF[j]
            glo = pl.multiple_of(flo & ~(size - 1), size)
            rdma = pltpu.make_async_remote_copy(
                src_ref=gbuf.at[pl.ds(glo, size), :],
                dst_ref=gbuf.at[pl.ds(glo, size), :],
                send_sem=ag_send.at[idx],
                recv_sem=ag_recv.at[idx],
                device_id=(my ^ (1 << j),),
                device_id_type=pl.DeviceIdType.MESH,
            )
            rdma.start()
            rdma.wait_recv()
            send_waits.append(rdma)

        for b in range(B):
            out_ref[b] = gbuf[b * SQ:(b + 1) * SQ, :].astype(jnp.float32)
        for rd in send_waits:
            rd.wait_send()

    out = pl.pallas_call(
        body,
        out_shape=jax.ShapeDtypeStruct((B, SQ, D_MODEL), jnp.float32),
        in_specs=[pl.BlockSpec(memory_space=pltpu.VMEM)] * 5,
        out_specs=pl.BlockSpec(memory_space=pltpu.VMEM),
        scratch_shapes=[
            pltpu.VMEM((ROWS, HD), jnp.bfloat16),
            pltpu.VMEM((ROWS, D_MODEL), jnp.bfloat16),
            pltpu.VMEM((RS_BUF_ROWS, D_MODEL), jnp.bfloat16),
            pltpu.VMEM((7, CHUNK, D_MODEL), jnp.bfloat16),
            pltpu.VMEM((ROWS, D_MODEL), jnp.bfloat16),
            pltpu.SemaphoreType.DMA((2,)),
            pltpu.SemaphoreType.DMA((2,)),
            pltpu.SemaphoreType.DMA((7,)),
            pltpu.SemaphoreType.DMA((7,)),
            pltpu.SemaphoreType.DMA((7,)),
            pltpu.SemaphoreType.DMA((7,)),
            pltpu.SemaphoreType.DMA((2,)),
            pltpu.SemaphoreType.DMA((2,)),
        ],
        compiler_params=pltpu.CompilerParams(collective_id=0),
    )(x, Wq, K_s, V_s, Wo)
    return out
